# baseline (speedup 1.0000x reference)
import functools

import jax
import jax.numpy as jnp
from jax import lax
from jax.experimental import pallas as pl
from jax.experimental.pallas import tpu as pltpu
from jax.experimental.pallas import tpu_sc as plsc

D = 768
FF = 2048
E = 8
K = 2
N = 2048
NK = N * K
BLK = 512
NB = NK // BLK + (E - 1)
TOTALPAD = NB * BLK
CH = 512


def _gate_route_body(x_ref, gw_ref, dest_ref, w_ref, counts_ref):
    x = x_ref[...]
    gw = gw_ref[...]
    logits = jax.lax.dot_general(x, gw, (((1,), (1,)), ((), ())),
                                 preferred_element_type=jnp.float32)
    m = jnp.max(logits, axis=1, keepdims=True)
    ex = jnp.exp(logits - m)
    s = ex / jnp.sum(ex, axis=1, keepdims=True)
    iota = jax.lax.broadcasted_iota(jnp.int32, s.shape, 1)
    m1 = jnp.max(s, axis=1, keepdims=True)
    i1 = jnp.min(jnp.where(s == m1, iota, E), axis=1, keepdims=True)
    s_masked = jnp.where(iota == i1, -jnp.inf, s)
    m2 = jnp.max(s_masked, axis=1, keepdims=True)
    i2 = jnp.min(jnp.where(s_masked == m2, iota, E), axis=1, keepdims=True)
    denom = m1 + m2 + 1e-20
    w_ref[...] = jnp.concatenate([m1 / denom, m2 / denom], axis=0)

    oh0 = (iota == i1).astype(jnp.float32)
    oh1 = (iota == i2).astype(jnp.float32)
    oh = jnp.concatenate([oh0, oh1], axis=0)
    rr = jax.lax.broadcasted_iota(jnp.int32, (CH, CH), 0)
    cc = jax.lax.broadcasted_iota(jnp.int32, (CH, CH), 1)
    tri = (cc < rr).astype(jnp.float32)
    carry = jnp.zeros((1, E), jnp.float32)
    ranks_parts = []
    for ci in range(NK // CH):
        blk = jax.lax.slice(oh, (ci * CH, 0), ((ci + 1) * CH, E))
        ranks_parts.append(
            jax.lax.dot_general(tri, blk, (((1,), (0,)), ((), ())),
                                preferred_element_type=jnp.float32) + carry)
        carry = carry + jnp.sum(blk, axis=0, keepdims=True)
    ranks = jnp.concatenate(ranks_parts, axis=0)
    pc = jnp.ceil(carry * (1.0 / BLK)) * BLK
    r8 = jax.lax.broadcasted_iota(jnp.int32, (E, E), 0)
    c8 = jax.lax.broadcasted_iota(jnp.int32, (E, E), 1)
    excl = (r8 < c8).astype(jnp.float32)
    pbase = jax.lax.dot_general(pc, excl, (((1,), (0,)), ((), ())),
                                preferred_element_type=jnp.float32)
    dest_f = jnp.sum(oh * (ranks + pbase), axis=1, keepdims=True)
    dest_ref[...] = dest_f.astype(jnp.int32)
    counts_ref[...] = carry


def _ffn_body(be_ref, act_ref, xs_ref, rw_ref, wg_ref, wu_ref, wd_ref,
              ys_ref):
    b = pl.program_id(0)

    @pl.when(act_ref[b] != 0)
    def _():
        x = xs_ref[...]
        g = jax.lax.dot_general(x, wg_ref[0], (((1,), (1,)), ((), ())),
                                preferred_element_type=jnp.float32)
        u = jax.lax.dot_general(x, wu_ref[0], (((1,), (1,)), ((), ())),
                                preferred_element_type=jnp.float32)
        hid = g * (1.0 / (1.0 + jnp.exp(-g))) * u
        out = jax.lax.dot_general(hid, wd_ref[0], (((1,), (1,)), ((), ())),
                                  preferred_element_type=jnp.float32)
        ys_ref[...] = out * rw_ref[...]


def _grouped_ffn(blk_expert, blk_act, xs, roww, w_gate, w_up, w_down):
    grid_spec = pltpu.PrefetchScalarGridSpec(
        num_scalar_prefetch=2,
        grid=(NB,),
        in_specs=[
            pl.BlockSpec((BLK, D), lambda b, be, act: (b, 0)),
            pl.BlockSpec((BLK, 1), lambda b, be, act: (b, 0)),
            pl.BlockSpec((1, FF, D), lambda b, be, act: (be[b], 0, 0)),
            pl.BlockSpec((1, FF, D), lambda b, be, act: (be[b], 0, 0)),
            pl.BlockSpec((1, D, FF), lambda b, be, act: (be[b], 0, 0)),
        ],
        out_specs=pl.BlockSpec((BLK, D), lambda b, be, act: (b, 0)),
    )
    return pl.pallas_call(
        _ffn_body,
        grid_spec=grid_spec,
        out_shape=jax.ShapeDtypeStruct((TOTALPAD, D), jnp.float32),
    )(blk_expert, blk_act, xs, roww, w_gate, w_up, w_down)


_SC_MESH = plsc.VectorSubcoreMesh(core_axis_name="c", subcore_axis_name="s")
NW = 32
SPW = NK // NW
TPW = N // NW


def _sc_dispatch(xf, dest, wflat):
    @functools.partial(
        pl.kernel,
        out_type=[
            jax.ShapeDtypeStruct((TOTALPAD, D), jnp.float32),
            jax.ShapeDtypeStruct((TOTALPAD,), jnp.float32),
        ],
        mesh=_SC_MESH,
        scratch_types=[
            pltpu.VMEM((2, SPW // 2), jnp.int32),
            pltpu.VMEM((SPW, D), jnp.float32),
            pltpu.VMEM((SPW,), jnp.float32),
            pltpu.SemaphoreType.DMA,
            pltpu.SemaphoreType.DMA,
        ],
    )
    def k(xf_hbm, dest_hbm, w_hbm, xs_hbm, rw_hbm, idx_v, rows_v, wv_v,
          sem0, sem1):
        wid = lax.axis_index("s") * 2 + lax.axis_index("c")
        slot0 = wid * SPW
        tok0 = lax.rem(slot0, N)
        H = SPW // 2
        cps = []
        for c in range(2):
            pltpu.sync_copy(dest_hbm.at[pl.ds(slot0 + c * H, H)],
                            idx_v.at[c])
            pltpu.sync_copy(xf_hbm.at[pl.ds(tok0 + c * H, H)],
                            rows_v.at[pl.ds(c * H, H), :])
            cps.append(pltpu.async_copy(rows_v.at[pl.ds(c * H, H), :],
                                        xs_hbm.at[idx_v.at[c]], sem0))
        pltpu.sync_copy(w_hbm.at[pl.ds(slot0, SPW)], wv_v)
        for c in range(2):
            cps.append(pltpu.async_copy(wv_v.at[pl.ds(c * H, H)],
                                        rw_hbm.at[idx_v.at[c]], sem1))
        for cp in cps:
            cp.wait()

    return k(xf, dest, wflat)


def _sc_combine(ys, dest):
    @functools.partial(
        pl.kernel,
        out_type=jax.ShapeDtypeStruct((N, D), jnp.float32),
        mesh=_SC_MESH,
        scratch_types=[
            pltpu.VMEM((TPW,), jnp.int32),
            pltpu.VMEM((TPW,), jnp.int32),
            pltpu.VMEM((TPW, D), jnp.float32),
            pltpu.VMEM((TPW, D), jnp.float32),
            pltpu.SemaphoreType.DMA,
        ],
    )
    def k(ys_hbm, dest_hbm, y_hbm, idx0_v, idx1_v, rows0_v, rows1_v, sem):
        wid = lax.axis_index("s") * 2 + lax.axis_index("c")
        t0 = wid * TPW
        pltpu.sync_copy(dest_hbm.at[pl.ds(t0, TPW)], idx0_v)
        pltpu.sync_copy(dest_hbm.at[pl.ds(N + t0, TPW)], idx1_v)
        cp0 = pltpu.async_copy(ys_hbm.at[idx0_v], rows0_v, sem)
        cp1 = pltpu.async_copy(ys_hbm.at[idx1_v], rows1_v, sem)
        cp0.wait()
        cp1.wait()

        def body(i, _):
            for j in range(D // 16):
                sl = pl.ds(j * 16, 16)
                rows0_v[i, sl] = rows0_v[i, sl] + rows1_v[i, sl]
            return 0

        lax.fori_loop(0, TPW, body, 0)
        pltpu.sync_copy(rows0_v, y_hbm.at[pl.ds(t0, TPW)])

    return k(ys, dest)


def kernel(x, gate_weight, w_gate, w_up, w_down):
    bsz, seq_len, h = x.shape
    xf = x.reshape(-1, h)

    dest2d, wflat2d, counts = pl.pallas_call(
        _gate_route_body,
        out_shape=[
            jax.ShapeDtypeStruct((NK, 1), jnp.int32),
            jax.ShapeDtypeStruct((NK, 1), jnp.float32),
            jax.ShapeDtypeStruct((1, E), jnp.float32),
        ],
    )(xf, gate_weight)
    dest = dest2d[:, 0]
    wflat = wflat2d[:, 0]

    pcb = jnp.ceil(counts[0] * (1.0 / BLK)).astype(jnp.int32)
    starts = jnp.cumsum(pcb) - pcb
    bids = jnp.arange(NB, dtype=jnp.int32)
    blk_expert = jnp.sum((starts[None, :] <= bids[:, None]).astype(jnp.int32),
                         axis=1) - 1
    used = jnp.sum(pcb)
    first = jnp.concatenate([jnp.ones((1,), jnp.bool_),
                             blk_expert[1:] != blk_expert[:-1]])
    blk_act = jnp.where(bids < used,
                        jnp.where(first, 2, 1), 0).astype(jnp.int32)

    xs, roww = _sc_dispatch(xf, dest, wflat)
    ys = _grouped_ffn(blk_expert, blk_act, xs, roww.reshape(TOTALPAD, 1),
                      w_gate, w_up, w_down)
    y = _sc_combine(ys, dest)
    return y.reshape(bsz, seq_len, h)


# speedup vs baseline: 1.0068x; 1.0068x over previous
import functools

import jax
import jax.numpy as jnp
from jax import lax
from jax.experimental import pallas as pl
from jax.experimental.pallas import tpu as pltpu
from jax.experimental.pallas import tpu_sc as plsc

D = 768
FF = 2048
E = 8
K = 2
N = 2048
NK = N * K
BLK = 512
NB = NK // BLK + (E - 1)
TOTALPAD = NB * BLK
CH = 512


def _gate_route_body(x_ref, gw_ref, dest_ref, w_ref, counts_ref):
    x = x_ref[...]
    gw = gw_ref[...]
    logits = jax.lax.dot_general(x, gw, (((1,), (1,)), ((), ())),
                                 preferred_element_type=jnp.float32)
    m = jnp.max(logits, axis=1, keepdims=True)
    ex = jnp.exp(logits - m)
    s = ex / jnp.sum(ex, axis=1, keepdims=True)
    iota = jax.lax.broadcasted_iota(jnp.int32, s.shape, 1)
    m1 = jnp.max(s, axis=1, keepdims=True)
    i1 = jnp.min(jnp.where(s == m1, iota, E), axis=1, keepdims=True)
    s_masked = jnp.where(iota == i1, -jnp.inf, s)
    m2 = jnp.max(s_masked, axis=1, keepdims=True)
    i2 = jnp.min(jnp.where(s_masked == m2, iota, E), axis=1, keepdims=True)
    denom = m1 + m2 + 1e-20
    w_ref[...] = jnp.concatenate([m1 / denom, m2 / denom], axis=0)

    oh0 = (iota == i1).astype(jnp.float32)
    oh1 = (iota == i2).astype(jnp.float32)
    oh = jnp.concatenate([oh0, oh1], axis=0)
    rr = jax.lax.broadcasted_iota(jnp.int32, (CH, CH), 0)
    cc = jax.lax.broadcasted_iota(jnp.int32, (CH, CH), 1)
    tri = (cc < rr).astype(jnp.float32)
    carry = jnp.zeros((1, E), jnp.float32)
    ranks_parts = []
    for ci in range(NK // CH):
        blk = jax.lax.slice(oh, (ci * CH, 0), ((ci + 1) * CH, E))
        ranks_parts.append(
            jax.lax.dot_general(tri, blk, (((1,), (0,)), ((), ())),
                                preferred_element_type=jnp.float32) + carry)
        carry = carry + jnp.sum(blk, axis=0, keepdims=True)
    ranks = jnp.concatenate(ranks_parts, axis=0)
    pc = jnp.ceil(carry * (1.0 / BLK)) * BLK
    r8 = jax.lax.broadcasted_iota(jnp.int32, (E, E), 0)
    c8 = jax.lax.broadcasted_iota(jnp.int32, (E, E), 1)
    excl = (r8 < c8).astype(jnp.float32)
    pbase = jax.lax.dot_general(pc, excl, (((1,), (0,)), ((), ())),
                                preferred_element_type=jnp.float32)
    dest_f = jnp.sum(oh * (ranks + pbase), axis=1, keepdims=True)
    dest_ref[...] = dest_f.astype(jnp.int32)
    counts_ref[...] = carry


def _ffn_body(be_ref, act_ref, xs_ref, rw_ref, wg_ref, wu_ref, wd_ref,
              ys_ref):
    b = pl.program_id(0)

    @pl.when(act_ref[b] != 0)
    def _():
        x = xs_ref[...]
        g = jax.lax.dot_general(x, wg_ref[0], (((1,), (1,)), ((), ())),
                                preferred_element_type=jnp.float32)
        u = jax.lax.dot_general(x, wu_ref[0], (((1,), (1,)), ((), ())),
                                preferred_element_type=jnp.float32)
        hid = g * (1.0 / (1.0 + jnp.exp(-g))) * u
        out = jax.lax.dot_general(hid, wd_ref[0], (((1,), (1,)), ((), ())),
                                  preferred_element_type=jnp.float32)
        ys_ref[...] = out * rw_ref[...]


def _grouped_ffn(blk_expert, blk_act, xs, roww, w_gate, w_up, w_down):
    grid_spec = pltpu.PrefetchScalarGridSpec(
        num_scalar_prefetch=2,
        grid=(NB,),
        in_specs=[
            pl.BlockSpec((BLK, D), lambda b, be, act: (b, 0)),
            pl.BlockSpec((BLK, 1), lambda b, be, act: (b, 0)),
            pl.BlockSpec((1, FF, D), lambda b, be, act: (be[b], 0, 0)),
            pl.BlockSpec((1, FF, D), lambda b, be, act: (be[b], 0, 0)),
            pl.BlockSpec((1, D, FF), lambda b, be, act: (be[b], 0, 0)),
        ],
        out_specs=pl.BlockSpec((BLK, D), lambda b, be, act: (b, 0)),
    )
    return pl.pallas_call(
        _ffn_body,
        grid_spec=grid_spec,
        out_shape=jax.ShapeDtypeStruct((TOTALPAD, D), jnp.float32),
    )(blk_expert, blk_act, xs, roww, w_gate, w_up, w_down)


_SC_MESH = plsc.VectorSubcoreMesh(core_axis_name="c", subcore_axis_name="s")
NW = 32
SPW = NK // NW
TPW = N // NW


def _sc_dispatch(xf, dest, wflat):
    @functools.partial(
        pl.kernel,
        out_type=[
            jax.ShapeDtypeStruct((TOTALPAD, D), jnp.float32),
            jax.ShapeDtypeStruct((TOTALPAD,), jnp.float32),
        ],
        mesh=_SC_MESH,
        scratch_types=[
            pltpu.VMEM((SPW,), jnp.int32),
            pltpu.VMEM((SPW, D), jnp.float32),
            pltpu.VMEM((SPW,), jnp.float32),
            pltpu.SemaphoreType.DMA,
            pltpu.SemaphoreType.DMA,
        ],
    )
    def k(xf_hbm, dest_hbm, w_hbm, xs_hbm, rw_hbm, idx_v, rows_v, wv_v,
          sem0, sem1):
        wid = lax.axis_index("s") * 2 + lax.axis_index("c")
        slot0 = wid * SPW
        tok0 = lax.rem(slot0, N)
        pltpu.sync_copy(dest_hbm.at[pl.ds(slot0, SPW)], idx_v)
        pltpu.sync_copy(xf_hbm.at[pl.ds(tok0, SPW)], rows_v)
        pltpu.sync_copy(w_hbm.at[pl.ds(slot0, SPW)], wv_v)
        cp0 = pltpu.async_copy(rows_v, xs_hbm.at[idx_v], sem0)
        cp1 = pltpu.async_copy(wv_v, rw_hbm.at[idx_v], sem1)
        cp0.wait()
        cp1.wait()

    return k(xf, dest, wflat)


def _sc_combine(ys, dest):
    @functools.partial(
        pl.kernel,
        out_type=jax.ShapeDtypeStruct((N, D), jnp.float32),
        mesh=_SC_MESH,
        scratch_types=[
            pltpu.VMEM((TPW,), jnp.int32),
            pltpu.VMEM((TPW,), jnp.int32),
            pltpu.VMEM((TPW, D), jnp.float32),
            pltpu.VMEM((TPW, D), jnp.float32),
            pltpu.SemaphoreType.DMA,
        ],
    )
    def k(ys_hbm, dest_hbm, y_hbm, idx0_v, idx1_v, rows0_v, rows1_v, sem):
        wid = lax.axis_index("s") * 2 + lax.axis_index("c")
        t0 = wid * TPW
        pltpu.sync_copy(dest_hbm.at[pl.ds(t0, TPW)], idx0_v)
        pltpu.sync_copy(dest_hbm.at[pl.ds(N + t0, TPW)], idx1_v)
        cp0 = pltpu.async_copy(ys_hbm.at[idx0_v], rows0_v, sem)
        cp1 = pltpu.async_copy(ys_hbm.at[idx1_v], rows1_v, sem)
        cp0.wait()
        cp1.wait()

        def body(i, _):
            for j in range(D // 16):
                sl = pl.ds(j * 16, 16)
                rows0_v[i, sl] = rows0_v[i, sl] + rows1_v[i, sl]
            return 0

        lax.fori_loop(0, TPW, body, 0)
        pltpu.sync_copy(rows0_v, y_hbm.at[pl.ds(t0, TPW)])

    return k(ys, dest)


def kernel(x, gate_weight, w_gate, w_up, w_down):
    bsz, seq_len, h = x.shape
    xf = x.reshape(-1, h)

    dest2d, wflat2d, counts = pl.pallas_call(
        _gate_route_body,
        out_shape=[
            jax.ShapeDtypeStruct((NK, 1), jnp.int32),
            jax.ShapeDtypeStruct((NK, 1), jnp.float32),
            jax.ShapeDtypeStruct((1, E), jnp.float32),
        ],
    )(xf, gate_weight)
    dest = dest2d[:, 0]
    wflat = wflat2d[:, 0]

    pcb = jnp.ceil(counts[0] * (1.0 / BLK)).astype(jnp.int32)
    starts = jnp.cumsum(pcb) - pcb
    bids = jnp.arange(NB, dtype=jnp.int32)
    blk_expert = jnp.sum((starts[None, :] <= bids[:, None]).astype(jnp.int32),
                         axis=1) - 1
    used = jnp.sum(pcb)
    first = jnp.concatenate([jnp.ones((1,), jnp.bool_),
                             blk_expert[1:] != blk_expert[:-1]])
    blk_act = jnp.where(bids < used,
                        jnp.where(first, 2, 1), 0).astype(jnp.int32)

    xs, roww = _sc_dispatch(xf, dest, wflat)
    ys = _grouped_ffn(blk_expert, blk_act, xs, roww.reshape(TOTALPAD, 1),
                      w_gate, w_up, w_down)
    y = _sc_combine(ys, dest)
    return y.reshape(bsz, seq_len, h)
